# Initial kernel scaffold; baseline (speedup 1.0000x reference)
import jax
import jax.numpy as jnp
from jax.experimental import pallas as pl


def kernel(img, trf):
    raise NotImplementedError("write your pallas kernel here")


# speedup vs baseline: 1.7746x; 1.7746x over previous
import functools

import jax
import jax.numpy as jnp
from jax import lax
from jax.experimental import pallas as pl
from jax.experimental.pallas import tpu as pltpu
from jax.experimental.pallas import tpu_sc as plsc

_B, _H, _W, _C = 2, 384, 384, 96
_HW = _H * _W
_P = _B * _HW
_N = 128
_NC, _NS = 2, 16
_NW = _NC * _NS
_CPB = _HW // _N
_CPW = _CPB // _NW
_NB = _C // 16

_GATHER_DNUMS = lax.GatherDimensionNumbers(
    offset_dims=(), collapsed_slice_dims=(0,), start_index_map=(0,))


def _lane_bcast(vec, lane):
    lidx = jnp.full((16, 1), lane, jnp.int32)
    return lax.gather(vec, lidx, _GATHER_DNUMS, (1,),
                      mode=lax.GatherScatterMode.PROMISE_IN_BOUNDS)


def _warp_body(img_hbm, dxy_hbm, gij_hbm, out_hbm,
               dxy_v, gij_v, i0_v, i1_v, i2_v, i3_v,
               w0_v, w1_v, w2_v, w3_v,
               g0_v, g1_v, g2_v, g3_v, out_v, sem_in, sem_g):
    wid = lax.axis_index("s") * _NC + lax.axis_index("c")

    for b in range(_B):
        boff = b * _HW

        def chunk_body(k, _, boff=boff):
            base_in = (k * _NW + wid) * _N
            base = boff + base_in
            cin0 = pltpu.make_async_copy(
                dxy_hbm.at[:, pl.ds(base, _N)], dxy_v, sem_in)
            cin0.start()
            cin1 = pltpu.make_async_copy(
                gij_hbm.at[:, pl.ds(base_in, _N)], gij_v, sem_in)
            cin1.start()
            cin0.wait()
            cin1.wait()

            for g in range(_N // 16):
                sl = pl.ds(g * 16, 16)
                cx = jnp.minimum(jnp.maximum(gij_v[0, sl] + dxy_v[0, sl], 0.0),
                                 float(_H - 1))
                cy = jnp.minimum(jnp.maximum(gij_v[1, sl] + dxy_v[1, sl], 0.0),
                                 float(_W - 1))
                xb = jnp.minimum(cx.astype(jnp.int32), _H - 2)
                yb = jnp.minimum(cy.astype(jnp.int32), _W - 2)
                fx = cx - xb.astype(jnp.float32)
                fy = cy - yb.astype(jnp.float32)
                rb = xb * _W + yb + boff
                i0_v[sl] = rb
                i1_v[sl] = rb + 1
                i2_v[sl] = rb + _W
                i3_v[sl] = rb + _W + 1
                wxa = 1.0 - fx
                wya = 1.0 - fy
                w0_v[sl] = wxa * wya
                w1_v[sl] = wxa * fy
                w2_v[sl] = fx * wya
                w3_v[sl] = fx * fy

            c0 = pltpu.make_async_copy(img_hbm.at[i0_v], g0_v, sem_g)
            c1 = pltpu.make_async_copy(img_hbm.at[i1_v], g1_v, sem_g)
            c2 = pltpu.make_async_copy(img_hbm.at[i2_v], g2_v, sem_g)
            c3 = pltpu.make_async_copy(img_hbm.at[i3_v], g3_v, sem_g)
            c0.start(); c1.start(); c2.start(); c3.start()
            c0.wait(); c1.wait(); c2.wait(); c3.wait()

            def grp_body(gq, _):
                base16 = gq * 16
                w0g = w0_v[pl.ds(base16, 16)]
                w1g = w1_v[pl.ds(base16, 16)]
                w2g = w2_v[pl.ds(base16, 16)]
                w3g = w3_v[pl.ds(base16, 16)]
                for li in range(16):
                    p = base16 + li
                    w0 = _lane_bcast(w0g, li)
                    w1 = _lane_bcast(w1g, li)
                    w2 = _lane_bcast(w2g, li)
                    w3 = _lane_bcast(w3g, li)
                    for blk in range(_NB):
                        csl = pl.ds(blk * 16, 16)
                        out_v[p, csl] = (w0 * g0_v[p, csl] + w1 * g1_v[p, csl]
                                         + w2 * g2_v[p, csl]
                                         + w3 * g3_v[p, csl])
                return 0

            lax.fori_loop(0, _N // 16, grp_body, 0)
            pltpu.sync_copy(out_v, out_hbm.at[pl.ds(base, _N)])
            return 0

        lax.fori_loop(0, _CPW, chunk_body, 0)


_warp = functools.partial(
    pl.kernel,
    out_type=jax.ShapeDtypeStruct((_P, _C), jnp.float32),
    mesh=plsc.VectorSubcoreMesh(core_axis_name="c", subcore_axis_name="s"),
    compiler_params=pltpu.CompilerParams(use_tc_tiling_on_sc=False),
    scratch_types=[
        pltpu.VMEM((2, _N), jnp.float32),
        pltpu.VMEM((2, _N), jnp.float32),
        pltpu.VMEM((_N,), jnp.int32),
        pltpu.VMEM((_N,), jnp.int32),
        pltpu.VMEM((_N,), jnp.int32),
        pltpu.VMEM((_N,), jnp.int32),
        pltpu.VMEM((_N,), jnp.float32),
        pltpu.VMEM((_N,), jnp.float32),
        pltpu.VMEM((_N,), jnp.float32),
        pltpu.VMEM((_N,), jnp.float32),
        pltpu.VMEM((_N, _C), jnp.float32),
        pltpu.VMEM((_N, _C), jnp.float32),
        pltpu.VMEM((_N, _C), jnp.float32),
        pltpu.VMEM((_N, _C), jnp.float32),
        pltpu.VMEM((_N, _C), jnp.float32),
        pltpu.SemaphoreType.DMA,
        pltpu.SemaphoreType.DMA,
    ],
)(_warp_body)


def kernel(img, trf):
    img2 = img.reshape(_P, _C)
    dxy = jnp.stack([trf[..., 0].reshape(-1), trf[..., 1].reshape(-1)])
    gi = lax.broadcasted_iota(jnp.float32, (_H, _W), 0).reshape(-1)
    gj = lax.broadcasted_iota(jnp.float32, (_H, _W), 1).reshape(-1)
    gij = jnp.stack([gi, gj])
    out2 = _warp(img2, dxy, gij)
    return out2.reshape(_B, _H, _W, _C), trf


# speedup vs baseline: 1.8893x; 1.0646x over previous
import functools

import jax
import jax.numpy as jnp
from jax import lax
from jax.experimental import pallas as pl
from jax.experimental.pallas import tpu as pltpu
from jax.experimental.pallas import tpu_sc as plsc

_B, _H, _W, _C = 2, 384, 384, 96
_HW = _H * _W
_P = _B * _HW
_N = 128
_NC, _NS = 2, 16
_NW = _NC * _NS
_CHUNKS = _P // _N
_CPW = _CHUNKS // _NW
_NB = _C // 16
_RB = 8
_NIDX = _CHUNKS * 4 * _N


def _prep_body(trf_ref, idx_ref, w_ref):
    b = pl.program_id(0)
    r = pl.program_id(1)
    t = trf_ref[0]
    dx = t[:, :, 0]
    dy = t[:, :, 1]
    gi = ((lax.broadcasted_iota(jnp.int32, (_RB, _W), 0)
           + r * _RB).astype(jnp.float32))
    gj = lax.broadcasted_iota(jnp.int32, (_RB, _W), 1).astype(jnp.float32)
    cx = jnp.minimum(jnp.maximum(gi + dx, 0.0), float(_H - 1))
    cy = jnp.minimum(jnp.maximum(gj + dy, 0.0), float(_W - 1))
    xb = jnp.minimum(cx.astype(jnp.int32), _H - 2)
    yb = jnp.minimum(cy.astype(jnp.int32), _W - 2)
    fx = cx - xb.astype(jnp.float32)
    fy = cy - yb.astype(jnp.float32)
    rb = xb * _W + yb + b * _HW
    nch = _RB * _W // _N
    idx4 = jnp.stack([c.reshape(nch, _N)
                      for c in (rb, rb + 1, rb + _W, rb + _W + 1)], axis=1)
    idx_ref[...] = idx4.reshape(nch * 4, _N)
    wxa = 1.0 - fx
    wya = 1.0 - fy
    w4 = jnp.stack([w.reshape(nch, _N)
                    for w in (wxa * wya, wxa * fy, fx * wya, fx * fy)],
                   axis=1)
    w_ref[...] = w4.reshape(nch * 4, _N)


_prep = pl.pallas_call(
    _prep_body,
    grid=(_B, _H // _RB),
    in_specs=[pl.BlockSpec((1, _RB, _W, 2), lambda b, r: (b, r, 0, 0))],
    out_specs=[
        pl.BlockSpec((_RB * _W * 4 // _N, _N),
                     lambda b, r: (b * (_H // _RB) + r, 0)),
        pl.BlockSpec((_RB * _W * 4 // _N, _N),
                     lambda b, r: (b * (_H // _RB) + r, 0)),
    ],
    out_shape=[
        jax.ShapeDtypeStruct((_NIDX // _N, _N), jnp.int32),
        jax.ShapeDtypeStruct((_NIDX // _N, _N), jnp.float32),
    ],
)


def _warp_body(img_hbm, idx_hbm, w_hbm, out_hbm,
               idx_v, w_v, g_v, out_v, sem_in, sem_g, sem_out):
    wid = lax.axis_index("s") * _NC + lax.axis_index("c")

    def chunk_of(i):
        return i * _NW + wid

    def in_copies(i, s):
        c = chunk_of(i)
        return (
            pltpu.make_async_copy(
                idx_hbm.at[pl.ds(c * 4, 4)], idx_v[s], sem_in[s]),
            pltpu.make_async_copy(
                w_hbm.at[pl.ds(c * 4, 4)], w_v[s], sem_in[s]),
        )

    def gather_copies(i, s):
        return tuple(
            pltpu.make_async_copy(
                img_hbm.at[idx_v[s].at[k]], g_v[s][k], sem_g[s])
            for k in range(4))

    def out_copy(i):
        c = chunk_of(i)
        return pltpu.make_async_copy(
            out_v, out_hbm.at[pl.ds(c * _N, _N)], sem_out)

    def fire(copies):
        for cp in copies:
            cp.start()

    def drain(copies):
        for cp in copies:
            cp.wait()

    def combine(i, s):
        gs = g_v[s]

        def grp_body(gq, _):
            b16 = gq * 16
            w0g = w_v[s][0, pl.ds(b16, 16)]
            w1g = w_v[s][1, pl.ds(b16, 16)]
            w2g = w_v[s][2, pl.ds(b16, 16)]
            w3g = w_v[s][3, pl.ds(b16, 16)]
            for li in range(16):
                p = b16 + li
                w0 = _lane_bcast(w0g, li)
                w1 = _lane_bcast(w1g, li)
                w2 = _lane_bcast(w2g, li)
                w3 = _lane_bcast(w3g, li)
                for blk in range(_NB):
                    csl = pl.ds(blk * 16, 16)
                    out_v[p, csl] = (
                        w0 * gs[0][p, csl] + w1 * gs[1][p, csl]
                        + w2 * gs[2][p, csl] + w3 * gs[3][p, csl])
            return 0

        lax.fori_loop(0, _N // 16, grp_body, 0)

    fire(in_copies(0, 0))
    fire(in_copies(1, 1))
    drain(in_copies(0, 0))
    fire(gather_copies(0, 0))

    def pair_body(k, _):
        for s in (0, 1):
            i = k * 2 + s

            @pl.when(i < _CPW - 1)
            def _():
                drain(in_copies(i + 1, 1 - s))
                fire(gather_copies(i + 1, 1 - s))

            drain(gather_copies(i, s))

            @pl.when(i >= 1)
            def _():
                out_copy(i - 1).wait()

            combine(i, s)
            out_copy(i).start()

            @pl.when(i < _CPW - 2)
            def _():
                fire(in_copies(i + 2, s))
        return 0

    lax.fori_loop(0, _CPW // 2, pair_body, 0)
    out_copy(_CPW - 1).wait()


_GATHER_DNUMS = lax.GatherDimensionNumbers(
    offset_dims=(), collapsed_slice_dims=(0,), start_index_map=(0,))


def _lane_bcast(vec, lane):
    lidx = jnp.full((16, 1), lane, jnp.int32)
    return lax.gather(vec, lidx, _GATHER_DNUMS, (1,),
                      mode=lax.GatherScatterMode.PROMISE_IN_BOUNDS)


_warp = functools.partial(
    pl.kernel,
    out_type=jax.ShapeDtypeStruct((_P, _C), jnp.float32),
    mesh=plsc.VectorSubcoreMesh(core_axis_name="c", subcore_axis_name="s"),
    compiler_params=pltpu.CompilerParams(use_tc_tiling_on_sc=False),
    scratch_types=[
        [pltpu.VMEM((4, _N), jnp.int32) for _ in range(2)],
        [pltpu.VMEM((4, _N), jnp.float32) for _ in range(2)],
        [[pltpu.VMEM((_N, _C), jnp.float32) for _ in range(4)]
         for _ in range(2)],
        pltpu.VMEM((_N, _C), jnp.float32),
        [pltpu.SemaphoreType.DMA for _ in range(2)],
        [pltpu.SemaphoreType.DMA for _ in range(2)],
        pltpu.SemaphoreType.DMA,
    ],
)(_warp_body)


def kernel(img, trf):
    img2 = img.reshape(_P, _C)
    idx1d, w1d = _prep(trf)
    out2 = _warp(img2, idx1d, w1d)
    return out2.reshape(_B, _H, _W, _C), trf


# speedup vs baseline: 2.5034x; 1.3251x over previous
import functools

import jax
import jax.numpy as jnp
from jax import lax
from jax.experimental import pallas as pl
from jax.experimental.pallas import tpu as pltpu
from jax.experimental.pallas import tpu_sc as plsc

_B, _H, _W, _C = 2, 384, 384, 96
_HW = _H * _W
_P = _B * _HW
_CP = 128
_N = 64
_NC, _NS = 2, 16
_NW = _NC * _NS
_CHUNKS = _P // _N
_CPW = _CHUNKS // _NW
_NB = _C // 16
_RB = 8
_NROW = _RB * _W


def _prep_body(trf_ref, idx_ref, w00_ref, w01_ref, w10_ref, w11_ref):
    b = pl.program_id(0)
    r = pl.program_id(1)
    t = trf_ref[0]
    dx = t[:, 0, :]
    dy = t[:, 1, :]
    gi = ((lax.broadcasted_iota(jnp.int32, (_RB, _W), 0)
           + r * _RB).astype(jnp.float32))
    gj = lax.broadcasted_iota(jnp.int32, (_RB, _W), 1).astype(jnp.float32)
    cx = jnp.minimum(jnp.maximum(gi + dx, 0.0), float(_H - 1))
    cy = jnp.minimum(jnp.maximum(gj + dy, 0.0), float(_W - 1))
    xb = jnp.minimum(cx.astype(jnp.int32), _H - 2)
    yb = jnp.minimum(cy.astype(jnp.int32), _W - 2)
    fx = cx - xb.astype(jnp.float32)
    fy = cy - yb.astype(jnp.float32)
    rb = xb * _W + yb + b * _HW
    nch = _NROW // _CP
    idx_ref[...] = rb.reshape(nch, _CP)
    wxa = 1.0 - fx
    wya = 1.0 - fy
    w00_ref[...] = (wxa * wya).reshape(nch, _CP)
    w01_ref[...] = (wxa * fy).reshape(nch, _CP)
    w10_ref[...] = (fx * wya).reshape(nch, _CP)
    w11_ref[...] = (fx * fy).reshape(nch, _CP)


_IDXROWS = _P // _CP


def _planar_spec():
    return pl.BlockSpec((_NROW // _CP, _CP),
                        lambda b, r: (b * (_H // _RB) + r, 0))


_prep = pl.pallas_call(
    _prep_body,
    grid=(_B, _H // _RB),
    in_specs=[pl.BlockSpec((1, _RB, 2, _W), lambda b, r: (b, r, 0, 0))],
    out_specs=[_planar_spec() for _ in range(5)],
    out_shape=[jax.ShapeDtypeStruct((_IDXROWS, _CP), jnp.int32)]
    + [jax.ShapeDtypeStruct((_IDXROWS, _CP), jnp.float32) for _ in range(4)],
)


def _pre_body(img_ref, tab_ref):
    x = img_ref[0]
    y = jnp.transpose(x, (0, 2, 1)).reshape(_NROW, _C)
    tab_ref[...] = jnp.concatenate(
        [y, jnp.zeros((_NROW, _CP - _C), jnp.float32)], axis=1)


_pre = pl.pallas_call(
    _pre_body,
    grid=(_B, _H // _RB),
    in_specs=[pl.BlockSpec((1, _RB, _C, _W), lambda b, r: (b, r, 0, 0))],
    out_specs=[pl.BlockSpec((_NROW, _CP),
                            lambda b, r: (b * (_H // _RB) + r, 0))],
    out_shape=[jax.ShapeDtypeStruct((_P, _CP), jnp.float32)],
)


def _post_body(tab_ref, img_ref):
    y = tab_ref[:, :_C]
    img_ref[0] = jnp.transpose(y.reshape(_RB, _W, _C), (0, 2, 1))


_post = pl.pallas_call(
    _post_body,
    grid=(_B, _H // _RB),
    in_specs=[pl.BlockSpec((_NROW, _CP),
                           lambda b, r: (b * (_H // _RB) + r, 0))],
    out_specs=[pl.BlockSpec((1, _RB, _C, _W), lambda b, r: (b, r, 0, 0))],
    out_shape=[jax.ShapeDtypeStruct((_B, _H, _C, _W), jnp.float32)],
)


def _warp_body(img_hbm, idxb_hbm, w00_hbm, w01_hbm, w10_hbm, w11_hbm,
               out_hbm, ib_v, idx_v, w_v, g_v, out_v, sem_in, sem_g, sem_out):
    wid = lax.axis_index("s") * _NC + lax.axis_index("c")
    w_hbms = (w00_hbm, w01_hbm, w10_hbm, w11_hbm)

    def chunk_of(i):
        return i * _NW + wid

    def in_copies(i, s):
        c = chunk_of(i)
        r2 = c // 2
        off = (c % 2) * _N
        cps = [pltpu.make_async_copy(
            idxb_hbm.at[r2, pl.ds(off, _N)], ib_v[s], sem_in[s])]
        for k in range(4):
            cps.append(pltpu.make_async_copy(
                w_hbms[k].at[r2, pl.ds(off, _N)], w_v[s].at[k], sem_in[s]))
        return cps

    def expand_idx(s):
        for g in range(_N // 16):
            sl = pl.ds(g * 16, 16)
            rv = ib_v[s][sl]
            idx_v[s][0, sl] = rv
            idx_v[s][1, sl] = rv + 1
            idx_v[s][2, sl] = rv + _W
            idx_v[s][3, sl] = rv + _W + 1

    def gather_copies(i, s):
        return tuple(
            pltpu.make_async_copy(
                img_hbm.at[idx_v[s].at[k]], g_v[s][k], sem_g[s])
            for k in range(4))

    def out_copy(i):
        c = chunk_of(i)
        return pltpu.make_async_copy(
            out_v, out_hbm.at[pl.ds(c * _N, _N)], sem_out)

    def fire(copies):
        for cp in copies:
            cp.start()

    def drain(copies):
        for cp in copies:
            cp.wait()

    def combine(i, s):
        gs = g_v[s]

        def grp_body(gq, _):
            b16 = gq * 16
            w0g = w_v[s][0, pl.ds(b16, 16)]
            w1g = w_v[s][1, pl.ds(b16, 16)]
            w2g = w_v[s][2, pl.ds(b16, 16)]
            w3g = w_v[s][3, pl.ds(b16, 16)]
            for li in range(16):
                p = b16 + li
                w0 = _lane_bcast(w0g, li)
                w1 = _lane_bcast(w1g, li)
                w2 = _lane_bcast(w2g, li)
                w3 = _lane_bcast(w3g, li)
                for blk in range(_NB):
                    csl = pl.ds(blk * 16, 16)
                    out_v[p, csl] = (
                        w0 * gs[0][p, csl] + w1 * gs[1][p, csl]
                        + w2 * gs[2][p, csl] + w3 * gs[3][p, csl])
            return 0

        lax.fori_loop(0, _N // 16, grp_body, 0)

    fire(in_copies(0, 0))
    fire(in_copies(1, 1))
    drain(in_copies(0, 0))
    expand_idx(0)
    fire(gather_copies(0, 0))

    def pair_body(k, _):
        for s in (0, 1):
            i = k * 2 + s

            @pl.when(i < _CPW - 1)
            def _():
                drain(in_copies(i + 1, 1 - s))
                expand_idx(1 - s)
                fire(gather_copies(i + 1, 1 - s))

            drain(gather_copies(i, s))

            @pl.when(i >= 1)
            def _():
                out_copy(i - 1).wait()

            combine(i, s)
            out_copy(i).start()

            @pl.when(i < _CPW - 2)
            def _():
                fire(in_copies(i + 2, s))
        return 0

    lax.fori_loop(0, _CPW // 2, pair_body, 0)
    out_copy(_CPW - 1).wait()


_GATHER_DNUMS = lax.GatherDimensionNumbers(
    offset_dims=(), collapsed_slice_dims=(0,), start_index_map=(0,))


def _lane_bcast(vec, lane):
    lidx = jnp.full((16, 1), lane, jnp.int32)
    return lax.gather(vec, lidx, _GATHER_DNUMS, (1,),
                      mode=lax.GatherScatterMode.PROMISE_IN_BOUNDS)


_warp = functools.partial(
    pl.kernel,
    out_type=jax.ShapeDtypeStruct((_P, _CP), jnp.float32),
    mesh=plsc.VectorSubcoreMesh(core_axis_name="c", subcore_axis_name="s"),
    compiler_params=pltpu.CompilerParams(use_tc_tiling_on_sc=False),
    scratch_types=[
        [pltpu.VMEM((_N,), jnp.int32) for _ in range(2)],
        [pltpu.VMEM((4, _N), jnp.int32) for _ in range(2)],
        [pltpu.VMEM((4, _N), jnp.float32) for _ in range(2)],
        [[pltpu.VMEM((_N, _CP), jnp.float32) for _ in range(4)]
         for _ in range(2)],
        pltpu.VMEM((_N, _CP), jnp.float32),
        [pltpu.SemaphoreType.DMA for _ in range(2)],
        [pltpu.SemaphoreType.DMA for _ in range(2)],
        pltpu.SemaphoreType.DMA,
    ],
)(_warp_body)


def kernel(img, trf):
    imgp, = _pre(jnp.transpose(img, (0, 1, 3, 2)))
    idxb, w00, w01, w10, w11 = _prep(jnp.transpose(trf, (0, 1, 3, 2)))
    outp = _warp(imgp, idxb, w00, w01, w10, w11)
    outt, = _post(outp)
    return jnp.transpose(outt, (0, 1, 3, 2)), trf
